# baseline (speedup 1.0000x reference)
import functools

import jax
import jax.numpy as jnp
from jax import lax
from jax.experimental import pallas as pl
from jax.experimental.pallas import tpu as pltpu
from jax.experimental.pallas import tpu_sc as plsc

N = 4096
NET = 4096
HID = 64
K = 20

NC = 2
NS = 16
NW = NC * NS
L = 16

RPW = N // NW
KPW = RPW * K

QG = 8
QNB = 3
NQG = RPW // QG

RPW2 = RPW // 2
KPW2 = RPW2 * K
SUB = 32
NSUB2 = RPW2 // SUB
KSUB = SUB * K
IDX_CHUNK = 128
NIC = KSUB // IDX_CHUNK
ND = HID // L


def _enc_body(xa_ref, xb_ref, xc_ref, xd_ref,
              w1a_ref, w1b_ref, w1c_ref, w1d_ref,
              b1_ref, w2_ref, b2_ref, w3_ref, b3_ref, h_ref):
    acc = jnp.dot(xa_ref[...].astype(jnp.bfloat16), w1a_ref[...],
                  preferred_element_type=jnp.float32)
    acc += jnp.dot(xb_ref[...].astype(jnp.bfloat16), w1b_ref[...],
                   preferred_element_type=jnp.float32)
    acc += jnp.dot(xc_ref[...].astype(jnp.bfloat16), w1c_ref[...],
                   preferred_element_type=jnp.float32)
    acc += jnp.dot(xd_ref[...].astype(jnp.bfloat16), w1d_ref[...],
                   preferred_element_type=jnp.float32)
    h1 = jnp.maximum(acc + b1_ref[...], 0.0)
    h2 = jnp.maximum(
        jnp.dot(h1, w2_ref[...], preferred_element_type=jnp.float32)
        + b2_ref[...], 0.0)
    h_ref[...] = jnp.maximum(
        jnp.dot(h2, w3_ref[...], preferred_element_type=jnp.float32)
        + b3_ref[...], 0.0)


def _encoder(X, W1, b1, W2, b2, W3, b3):
    BM = 256
    NS4 = NET // 4
    W1b = W1.astype(jnp.bfloat16)
    x_specs = [
        pl.BlockSpec((BM, NS4), lambda i, j=j: (i, j)) for j in range(4)]
    w_specs = [
        pl.BlockSpec((NS4, 256), lambda i, j=j: (j, 0)) for j in range(4)]
    return pl.pallas_call(
        _enc_body,
        grid=(N // BM,),
        in_specs=x_specs + w_specs + [
            pl.BlockSpec((1, 256), lambda i: (0, 0)),
            pl.BlockSpec((256, 84), lambda i: (0, 0)),
            pl.BlockSpec((1, 84), lambda i: (0, 0)),
            pl.BlockSpec((84, HID), lambda i: (0, 0)),
            pl.BlockSpec((1, HID), lambda i: (0, 0)),
        ],
        out_specs=pl.BlockSpec((BM, HID), lambda i: (i, 0)),
        out_shape=jax.ShapeDtypeStruct((N, HID), jnp.float32),
        compiler_params=pltpu.CompilerParams(
            dimension_semantics=("parallel",)),
    )(X, X, X, X, W1b, W1b, W1b, W1b, b1.reshape(1, -1),
      W2, b2.reshape(1, -1), W3, b3.reshape(1, -1))


def _dec_compute(z_ref, d1_ref, bd1_ref, d2_ref, bd2_ref, o_ref):
    hd = jnp.maximum(
        jnp.dot(z_ref[...], d1_ref[...], preferred_element_type=jnp.float32)
        + bd1_ref[...], 0.0)
    o_ref[...] = jnp.maximum(
        jnp.dot(hd, d2_ref[...], preferred_element_type=jnp.float32)
        + bd2_ref[...], 0.0)


def _dec_body_a(z_ref, d1_ref, bd1_ref, d2_ref, bd2_ref, o_ref):
    _dec_compute(z_ref, d1_ref, bd1_ref, d2_ref, bd2_ref, o_ref)


def _dec_body_b(z_ref, d1_ref, bd1_ref, d2_ref, bd2_ref, prev_ref, o_ref):
    del prev_ref
    _dec_compute(z_ref, d1_ref, bd1_ref, d2_ref, bd2_ref, o_ref)


_DEC_W_SPECS = [
    pl.BlockSpec((HID, 256), lambda i: (0, 0)),
    pl.BlockSpec((1, 256), lambda i: (0, 0)),
    pl.BlockSpec((256, NET), lambda i: (0, 0)),
    pl.BlockSpec((1, NET), lambda i: (0, 0)),
]
_DEC_BM = 512


def _decoder_a(Za, D1, bd1, D2, bd2):
    return pl.pallas_call(
        _dec_body_a,
        grid=(N // 2 // _DEC_BM,),
        in_specs=[pl.BlockSpec((_DEC_BM, HID), lambda i: (i, 0))]
        + _DEC_W_SPECS,
        out_specs=pl.BlockSpec((_DEC_BM, NET), lambda i: (i, 0)),
        out_shape=jax.ShapeDtypeStruct((N, NET), jnp.float32),
        compiler_params=pltpu.CompilerParams(
            dimension_semantics=("parallel",)),
    )(Za, D1, bd1.reshape(1, -1), D2, bd2.reshape(1, -1))


def _decoder_b(Zb, D1, bd1, D2, bd2, prev):
    nb = N // 2 // _DEC_BM
    return pl.pallas_call(
        _dec_body_b,
        grid=(nb,),
        in_specs=[pl.BlockSpec((_DEC_BM, HID), lambda i: (i, 0))]
        + _DEC_W_SPECS
        + [pl.BlockSpec(memory_space=pl.ANY)],
        out_specs=pl.BlockSpec((_DEC_BM, NET), lambda i, nb=nb: (i + nb, 0)),
        out_shape=jax.ShapeDtypeStruct((N, NET), jnp.float32),
        input_output_aliases={5: 0},
        compiler_params=pltpu.CompilerParams(
            dimension_semantics=("parallel",)),
    )(Zb, D1, bd1.reshape(1, -1), D2, bd2.reshape(1, -1), prev)


@functools.lru_cache(maxsize=None)
def _mesh():
    return plsc.VectorSubcoreMesh(
        core_axis_name="c", subcore_axis_name="s",
        num_cores=NC, num_subcores=NS)


_QW_SCRATCH = [
    pltpu.VMEM((KPW,), jnp.int32),
    pltpu.VMEM((KPW,), jnp.float32),
    pltpu.VMEM((QNB, QG, NET), jnp.float32),
    pltpu.SemaphoreType.DMA,
    pltpu.SemaphoreType.DMA,
    pltpu.SemaphoreType.DMA,
]


def _qw_body(q_hbm, topk_hbm, qw_hbm, topk_v, qw_v, qr_v, sem0, sem1, sem2):
    wid = lax.axis_index("s") * NC + lax.axis_index("c")
    base = wid * RPW
    kbase = base * K
    pltpu.sync_copy(topk_hbm.at[pl.ds(kbase, KPW)], topk_v)
    sems = (sem0, sem1, sem2)

    def fire(g, s):
        def _enq():
            pltpu.async_copy(
                q_hbm.at[pl.ds(base + g * QG, QG)], qr_v.at[s], sems[s])
        if isinstance(g, int):
            if g < NQG:
                _enq()
        else:
            pl.when(g < NQG)(_enq)

    def drain(s):
        pltpu.make_async_copy(
            q_hbm.at[pl.ds(base, QG)], qr_v.at[s], sems[s]).wait()

    def process(g, s):
        for n in range(QG):
            kb = g * QG * K + n * K
            tk0 = topk_v[pl.ds(kb, L)]
            tk1 = topk_v[pl.ds(kb + K - L, L)]
            row = jnp.full((L,), n, jnp.int32)
            g0 = plsc.load_gather(qr_v.at[s], [row, tk0])
            g1 = plsc.load_gather(qr_v.at[s], [row, tk1])
            qw_v[pl.ds(kb, L)] = g0
            qw_v[pl.ds(kb + K - L, L)] = g1

    for s in range(QNB):
        fire(s, s)

    def group_trip(i, carry):
        for s in range(QNB):
            g = QNB * i + s
            drain(s)
            process(g, s)
            fire(g + QNB, s)
        return carry
    lax.fori_loop(0, NQG // QNB, group_trip, 0)
    for g in range((NQG // QNB) * QNB, NQG):
        s = g % QNB
        drain(s)
        process(g, s)
    pltpu.sync_copy(qw_v, qw_hbm.at[pl.ds(kbase, KPW)])


@functools.lru_cache(maxsize=None)
def _qw_extract():
    return pl.kernel(
        _qw_body,
        out_type=jax.ShapeDtypeStruct((N * K,), jnp.float32),
        mesh=_mesh(),
        scratch_types=_QW_SCRATCH,
        compiler_params=pltpu.CompilerParams(needs_layout_passes=False),
    )


_AGG_SCRATCH = [
    pltpu.VMEM((KPW2,), jnp.int32),
    pltpu.VMEM((KPW2,), jnp.float32),
    pltpu.VMEM((RPW2, HID), jnp.float32),
    pltpu.VMEM((2, KSUB, HID), jnp.float32),
    pltpu.VMEM((SUB, HID), jnp.float32),
    pltpu.SemaphoreType.DMA,
    pltpu.SemaphoreType.DMA,
]


def _make_agg_body(h):
    def _agg_body(h_hbm, topk_hbm, qw_hbm, z_hbm,
                  topk_v, qw_v, hown_v, rows_v, z_v, sem0, sem1):
        wid = lax.axis_index("s") * NC + lax.axis_index("c")
        lbase = wid * RPW2
        gbase = h * (N // 2) + lbase
        kbase = gbase * K
        pltpu.sync_copy(topk_hbm.at[pl.ds(kbase, KPW2)], topk_v)
        pltpu.sync_copy(qw_hbm.at[pl.ds(kbase, KPW2)], qw_v)
        pltpu.sync_copy(h_hbm.at[pl.ds(gbase, RPW2)], hown_v)
        sems = (sem0, sem1)

        def fire(c, s):
            if c >= NSUB2:
                return
            cb = c * KSUB
            for j in range(NIC):
                pltpu.async_copy(
                    h_hbm.at[topk_v.at[pl.ds(cb + j * IDX_CHUNK, IDX_CHUNK)]],
                    rows_v.at[s, pl.ds(j * IDX_CHUNK, IDX_CHUNK)], sems[s])

        def drain(s):
            for j in range(NIC):
                pltpu.make_async_copy(
                    h_hbm.at[pl.ds(0, IDX_CHUNK)],
                    rows_v.at[s, pl.ds(j * IDX_CHUNK, IDX_CHUNK)],
                    sems[s]).wait()

        fire(0, 0)
        for c in range(NSUB2):
            s = c & 1
            drain(s)
            fire(c + 1, s ^ 1)

            def node_step(n, carry, s=s, c=c):
                node = c * SUB + n
                kb = n * K
                qa = qw_v[pl.ds(node * K, L)]
                qb = qw_v[pl.ds(node * K + K - L, L)]
                accs = [hown_v[node, pl.ds(d * L, L)] for d in range(ND)]
                for k in range(K):
                    if k < L:
                        b = jnp.broadcast_to(qa[k], (L,))
                    else:
                        b = jnp.broadcast_to(qb[k - (K - L)], (L,))
                    for d in range(ND):
                        accs[d] = (accs[d]
                                   + b * rows_v[s, kb + k, pl.ds(d * L, L)])
                for d in range(ND):
                    z_v[n, pl.ds(d * L, L)] = accs[d]
                return carry
            lax.fori_loop(0, SUB, node_step, 0)
            pltpu.sync_copy(z_v, z_hbm.at[pl.ds(lbase + c * SUB, SUB)])
    return _agg_body


@functools.lru_cache(maxsize=None)
def _agg_half(h):
    return pl.kernel(
        _make_agg_body(h),
        out_type=jax.ShapeDtypeStruct((N // 2, HID), jnp.float32),
        mesh=_mesh(),
        scratch_types=_AGG_SCRATCH,
        compiler_params=pltpu.CompilerParams(
            needs_layout_passes=False, use_tc_tiling_on_sc=False),
    )


def kernel(X, Q, top_k, W1, b1, W2, b2, W3, b3, D1, bd1, D2, bd2):
    H = _encoder(X, W1, b1, W2, b2, W3, b3)
    topk_flat = top_k.reshape(-1)
    qw = _qw_extract()(Q, topk_flat)
    Za = _agg_half(0)(H, topk_flat, qw)
    Zb = _agg_half(1)(H, topk_flat, qw)
    out_a = _decoder_a(Za, D1, bd1, D2, bd2)
    X_rec = _decoder_b(Zb, D1, bd1, D2, bd2, out_a)
    Z = jnp.concatenate([Za, Zb], axis=0)
    return (X_rec, Z)


# speedup vs baseline: 1.0017x; 1.0017x over previous
import functools

import jax
import jax.numpy as jnp
from jax import lax
from jax.experimental import pallas as pl
from jax.experimental.pallas import tpu as pltpu
from jax.experimental.pallas import tpu_sc as plsc

N = 4096
NET = 4096
HID = 64
K = 20

NC = 2
NS = 16
NW = NC * NS
L = 16

RPW = N // NW
KPW = RPW * K

QG = 8
QNB = 3
NQG = RPW // QG

RPW2 = RPW // 2
KPW2 = RPW2 * K
SUB = 32
NSUB2 = RPW2 // SUB
KSUB = SUB * K
IDX_CHUNK = 128
NIC = KSUB // IDX_CHUNK
ND = HID // L


def _enc_body(xa_ref, xb_ref, xc_ref, xd_ref,
              w1a_ref, w1b_ref, w1c_ref, w1d_ref,
              b1_ref, w2_ref, b2_ref, w3_ref, b3_ref, h_ref):
    acc = jnp.dot(xa_ref[...].astype(jnp.bfloat16), w1a_ref[...],
                  preferred_element_type=jnp.float32)
    acc += jnp.dot(xb_ref[...].astype(jnp.bfloat16), w1b_ref[...],
                   preferred_element_type=jnp.float32)
    acc += jnp.dot(xc_ref[...].astype(jnp.bfloat16), w1c_ref[...],
                   preferred_element_type=jnp.float32)
    acc += jnp.dot(xd_ref[...].astype(jnp.bfloat16), w1d_ref[...],
                   preferred_element_type=jnp.float32)
    h1 = jnp.maximum(acc + b1_ref[...], 0.0)
    h2 = jnp.maximum(
        jnp.dot(h1, w2_ref[...], preferred_element_type=jnp.float32)
        + b2_ref[...], 0.0)
    h_ref[...] = jnp.maximum(
        jnp.dot(h2, w3_ref[...], preferred_element_type=jnp.float32)
        + b3_ref[...], 0.0)


def _encoder(X, W1, b1, W2, b2, W3, b3):
    BM = 256
    NS4 = NET // 4
    W1b = W1.astype(jnp.bfloat16)
    x_specs = [
        pl.BlockSpec((BM, NS4), lambda i, j=j: (i, j)) for j in range(4)]
    w_specs = [
        pl.BlockSpec((NS4, 256), lambda i, j=j: (j, 0)) for j in range(4)]
    return pl.pallas_call(
        _enc_body,
        grid=(N // BM,),
        in_specs=x_specs + w_specs + [
            pl.BlockSpec((1, 256), lambda i: (0, 0)),
            pl.BlockSpec((256, 84), lambda i: (0, 0)),
            pl.BlockSpec((1, 84), lambda i: (0, 0)),
            pl.BlockSpec((84, HID), lambda i: (0, 0)),
            pl.BlockSpec((1, HID), lambda i: (0, 0)),
        ],
        out_specs=pl.BlockSpec((BM, HID), lambda i: (i, 0)),
        out_shape=jax.ShapeDtypeStruct((N, HID), jnp.float32),
        compiler_params=pltpu.CompilerParams(
            dimension_semantics=("parallel",)),
    )(X, X, X, X, W1b, W1b, W1b, W1b, b1.reshape(1, -1),
      W2, b2.reshape(1, -1), W3, b3.reshape(1, -1))


def _dec_compute(z_ref, d1_ref, bd1_ref, d2_ref, bd2_ref, o_ref):
    hd = jnp.maximum(
        jnp.dot(z_ref[...], d1_ref[...], preferred_element_type=jnp.float32)
        + bd1_ref[...], 0.0)
    o_ref[...] = jnp.maximum(
        jnp.dot(hd, d2_ref[...], preferred_element_type=jnp.float32)
        + bd2_ref[...], 0.0)


def _dec_body_a(z_ref, d1_ref, bd1_ref, d2_ref, bd2_ref, o_ref):
    _dec_compute(z_ref, d1_ref, bd1_ref, d2_ref, bd2_ref, o_ref)


def _dec_body_b(z_ref, d1_ref, bd1_ref, d2_ref, bd2_ref, prev_ref, o_ref):
    del prev_ref
    _dec_compute(z_ref, d1_ref, bd1_ref, d2_ref, bd2_ref, o_ref)


_DEC_W_SPECS = [
    pl.BlockSpec((HID, 256), lambda i: (0, 0)),
    pl.BlockSpec((1, 256), lambda i: (0, 0)),
    pl.BlockSpec((256, NET), lambda i: (0, 0)),
    pl.BlockSpec((1, NET), lambda i: (0, 0)),
]
_DEC_BM = 512


def _decoder_a(Za, D1, bd1, D2, bd2):
    return pl.pallas_call(
        _dec_body_a,
        grid=(N // 2 // _DEC_BM,),
        in_specs=[pl.BlockSpec((_DEC_BM, HID), lambda i: (i, 0))]
        + _DEC_W_SPECS,
        out_specs=pl.BlockSpec((_DEC_BM, NET), lambda i: (i, 0)),
        out_shape=jax.ShapeDtypeStruct((N, NET), jnp.float32),
        compiler_params=pltpu.CompilerParams(
            dimension_semantics=("parallel",)),
    )(Za, D1, bd1.reshape(1, -1), D2, bd2.reshape(1, -1))


def _decoder_b(Zb, D1, bd1, D2, bd2, prev):
    nb = N // 2 // _DEC_BM
    return pl.pallas_call(
        _dec_body_b,
        grid=(nb,),
        in_specs=[pl.BlockSpec((_DEC_BM, HID), lambda i: (i, 0))]
        + _DEC_W_SPECS
        + [pl.BlockSpec(memory_space=pl.ANY)],
        out_specs=pl.BlockSpec((_DEC_BM, NET), lambda i, nb=nb: (i + nb, 0)),
        out_shape=jax.ShapeDtypeStruct((N, NET), jnp.float32),
        input_output_aliases={5: 0},
        compiler_params=pltpu.CompilerParams(
            dimension_semantics=("parallel",)),
    )(Zb, D1, bd1.reshape(1, -1), D2, bd2.reshape(1, -1), prev)


@functools.lru_cache(maxsize=None)
def _mesh():
    return plsc.VectorSubcoreMesh(
        core_axis_name="c", subcore_axis_name="s",
        num_cores=NC, num_subcores=NS)


_QW_SCRATCH = [
    pltpu.VMEM((KPW,), jnp.int32),
    pltpu.VMEM((KPW,), jnp.float32),
    pltpu.VMEM((QNB, QG, NET), jnp.float32),
    pltpu.SemaphoreType.DMA,
    pltpu.SemaphoreType.DMA,
    pltpu.SemaphoreType.DMA,
]


def _qw_body(q_hbm, topk_hbm, qw_hbm, topk_v, qw_v, qr_v, sem0, sem1, sem2):
    wid = lax.axis_index("s") * NC + lax.axis_index("c")
    base = wid * RPW
    kbase = base * K
    pltpu.sync_copy(topk_hbm.at[pl.ds(kbase, KPW)], topk_v)
    sems = (sem0, sem1, sem2)

    def fire(g, s):
        def _enq():
            pltpu.async_copy(
                q_hbm.at[pl.ds(base + g * QG, QG)], qr_v.at[s], sems[s])
        if isinstance(g, int):
            if g < NQG:
                _enq()
        else:
            pl.when(g < NQG)(_enq)

    def drain(s):
        pltpu.make_async_copy(
            q_hbm.at[pl.ds(base, QG)], qr_v.at[s], sems[s]).wait()

    def process(g, s):
        for n in range(QG):
            kb = g * QG * K + n * K
            tk0 = topk_v[pl.ds(kb, L)]
            tk1 = topk_v[pl.ds(kb + K - L, L)]
            row = jnp.full((L,), n, jnp.int32)
            g0 = plsc.load_gather(qr_v.at[s], [row, tk0])
            g1 = plsc.load_gather(qr_v.at[s], [row, tk1])
            qw_v[pl.ds(kb, L)] = g0
            qw_v[pl.ds(kb + K - L, L)] = g1

    for s in range(QNB):
        fire(s, s)

    def group_trip(i, carry):
        for s in range(QNB):
            g = QNB * i + s
            drain(s)
            process(g, s)
            fire(g + QNB, s)
        return carry
    lax.fori_loop(0, NQG // QNB, group_trip, 0)
    for g in range((NQG // QNB) * QNB, NQG):
        s = g % QNB
        drain(s)
        process(g, s)
    pltpu.sync_copy(qw_v, qw_hbm.at[pl.ds(kbase, KPW)])


@functools.lru_cache(maxsize=None)
def _qw_extract():
    return pl.kernel(
        _qw_body,
        out_type=jax.ShapeDtypeStruct((N * K,), jnp.float32),
        mesh=_mesh(),
        scratch_types=_QW_SCRATCH,
        compiler_params=pltpu.CompilerParams(needs_layout_passes=False),
    )


_AGG_SCRATCH = [
    pltpu.VMEM((KPW2,), jnp.int32),
    pltpu.VMEM((KPW2,), jnp.float32),
    pltpu.VMEM((RPW2, HID), jnp.float32),
    pltpu.VMEM((2, KSUB, HID), jnp.float32),
    pltpu.VMEM((SUB, HID), jnp.float32),
    pltpu.SemaphoreType.DMA,
    pltpu.SemaphoreType.DMA,
]


def _make_agg_body(h):
    def _agg_body(h_hbm, topk_hbm, qw_hbm, z_hbm,
                  topk_v, qw_v, hown_v, rows_v, z_v, sem0, sem1):
        wid = lax.axis_index("s") * NC + lax.axis_index("c")
        lbase = wid * RPW2
        gbase = h * (N // 2) + lbase
        kbase = gbase * K
        pltpu.sync_copy(topk_hbm.at[pl.ds(kbase, KPW2)], topk_v)
        pltpu.sync_copy(qw_hbm.at[pl.ds(kbase, KPW2)], qw_v)
        pltpu.sync_copy(h_hbm.at[pl.ds(gbase, RPW2)], hown_v)
        sems = (sem0, sem1)

        def fire(c, s):
            if c >= NSUB2:
                return
            cb = c * KSUB
            for j in range(NIC):
                pltpu.async_copy(
                    h_hbm.at[topk_v.at[pl.ds(cb + j * IDX_CHUNK, IDX_CHUNK)]],
                    rows_v.at[s, pl.ds(j * IDX_CHUNK, IDX_CHUNK)], sems[s])

        def drain(s):
            for j in range(NIC):
                pltpu.make_async_copy(
                    h_hbm.at[pl.ds(0, IDX_CHUNK)],
                    rows_v.at[s, pl.ds(j * IDX_CHUNK, IDX_CHUNK)],
                    sems[s]).wait()

        fire(0, 0)
        for c in range(NSUB2):
            s = c & 1
            drain(s)
            fire(c + 1, s ^ 1)

            def node_step(n, carry, s=s, c=c):
                node = c * SUB + n
                kb = n * K
                qa = qw_v[pl.ds(node * K, L)]
                qb = qw_v[pl.ds(node * K + K - L, L)]
                accs = [hown_v[node, pl.ds(d * L, L)] for d in range(ND)]
                for k in range(K):
                    if k < L:
                        b = jnp.broadcast_to(qa[k], (L,))
                    else:
                        b = jnp.broadcast_to(qb[k - (K - L)], (L,))
                    for d in range(ND):
                        accs[d] = (accs[d]
                                   + b * rows_v[s, kb + k, pl.ds(d * L, L)])
                for d in range(ND):
                    z_v[n, pl.ds(d * L, L)] = accs[d]
                return carry
            lax.fori_loop(0, SUB, node_step, 0)
            pltpu.sync_copy(z_v, z_hbm.at[pl.ds(lbase + c * SUB, SUB)])
    return _agg_body


@functools.lru_cache(maxsize=None)
def _agg_half(h):
    return pl.kernel(
        _make_agg_body(h),
        out_type=jax.ShapeDtypeStruct((N // 2, HID), jnp.float32),
        mesh=_mesh(),
        scratch_types=_AGG_SCRATCH,
        compiler_params=pltpu.CompilerParams(
            needs_layout_passes=False, use_tc_tiling_on_sc=False),
    )


def kernel(X, Q, top_k, W1, b1, W2, b2, W3, b3, D1, bd1, D2, bd2):
    H = _encoder(X, W1, b1, W2, b2, W3, b3)
    topk_flat = top_k.reshape(-1)
    qw = _qw_extract()(Q, topk_flat)
    Za = _agg_half(0)(H, topk_flat, qw)
    out_a = _decoder_a(Za, D1, bd1, D2, bd2)
    Zb = _agg_half(1)(H, topk_flat, qw)
    X_rec = _decoder_b(Zb, D1, bd1, D2, bd2, out_a)
    Z = jnp.concatenate([Za, Zb], axis=0)
    return (X_rec, Z)


# speedup vs baseline: 1.1060x; 1.1041x over previous
import functools

import jax
import jax.numpy as jnp
from jax import lax
from jax.experimental import pallas as pl
from jax.experimental.pallas import tpu as pltpu
from jax.experimental.pallas import tpu_sc as plsc

N = 4096
NET = 4096
HID = 64
K = 20

NC = 2
NS = 16
NW = NC * NS
L = 16

RPW = N // NW
KPW = RPW * K

QG = 8
QNB = 3
NQG = RPW // QG

SUB = 32
NSUB = RPW // SUB
KSUB = SUB * K
IDX_CHUNK = 128
NIC = KSUB // IDX_CHUNK
ND = HID // L


def _enc_body(x_ref, w1_ref, b1_ref, w2_ref, b2_ref, w3_ref, b3_ref, h_ref):
    h1 = jnp.maximum(
        jnp.dot(x_ref[...].astype(jnp.bfloat16), w1_ref[...],
                preferred_element_type=jnp.float32) + b1_ref[...], 0.0)
    h2 = jnp.maximum(
        jnp.dot(h1, w2_ref[...], preferred_element_type=jnp.float32)
        + b2_ref[...], 0.0)
    h_ref[...] = jnp.maximum(
        jnp.dot(h2, w3_ref[...], preferred_element_type=jnp.float32)
        + b3_ref[...], 0.0)


def _encoder(X, W1, b1, W2, b2, W3, b3):
    BM = 256
    return pl.pallas_call(
        _enc_body,
        grid=(N // BM,),
        in_specs=[
            pl.BlockSpec((BM, NET), lambda i: (i, 0)),
            pl.BlockSpec((NET, 256), lambda i: (0, 0)),
            pl.BlockSpec((1, 256), lambda i: (0, 0)),
            pl.BlockSpec((256, 84), lambda i: (0, 0)),
            pl.BlockSpec((1, 84), lambda i: (0, 0)),
            pl.BlockSpec((84, HID), lambda i: (0, 0)),
            pl.BlockSpec((1, HID), lambda i: (0, 0)),
        ],
        out_specs=pl.BlockSpec((BM, HID), lambda i: (i, 0)),
        out_shape=jax.ShapeDtypeStruct((N, HID), jnp.float32),
        compiler_params=pltpu.CompilerParams(
            dimension_semantics=("parallel",)),
    )(X, W1.astype(jnp.bfloat16), b1.reshape(1, -1), W2, b2.reshape(1, -1),
      W3, b3.reshape(1, -1))


def _dec_body(z_ref, d1_ref, bd1_ref, d2_ref, bd2_ref, o_ref):
    hd = jnp.maximum(
        jnp.dot(z_ref[...], d1_ref[...], preferred_element_type=jnp.float32)
        + bd1_ref[...], 0.0)
    o_ref[...] = jnp.maximum(
        jnp.dot(hd, d2_ref[...], preferred_element_type=jnp.float32)
        + bd2_ref[...], 0.0)


def _decoder(Z, D1, bd1, D2, bd2):
    BM = 512
    return pl.pallas_call(
        _dec_body,
        grid=(N // BM,),
        in_specs=[
            pl.BlockSpec((BM, HID), lambda i: (i, 0)),
            pl.BlockSpec((HID, 256), lambda i: (0, 0)),
            pl.BlockSpec((1, 256), lambda i: (0, 0)),
            pl.BlockSpec((256, NET), lambda i: (0, 0)),
            pl.BlockSpec((1, NET), lambda i: (0, 0)),
        ],
        out_specs=pl.BlockSpec((BM, NET), lambda i: (i, 0)),
        out_shape=jax.ShapeDtypeStruct((N, NET), jnp.float32),
        compiler_params=pltpu.CompilerParams(
            dimension_semantics=("parallel",)),
    )(Z, D1, bd1.reshape(1, -1), D2, bd2.reshape(1, -1))


@functools.lru_cache(maxsize=None)
def _mesh():
    return plsc.VectorSubcoreMesh(
        core_axis_name="c", subcore_axis_name="s",
        num_cores=NC, num_subcores=NS)


_QW_SCRATCH = [
    pltpu.VMEM((KPW,), jnp.int32),
    pltpu.VMEM((KPW,), jnp.float32),
    pltpu.VMEM((QNB, QG, NET), jnp.float32),
    pltpu.SemaphoreType.DMA,
    pltpu.SemaphoreType.DMA,
    pltpu.SemaphoreType.DMA,
]


def _qw_body(q_hbm, topk_hbm, qw_hbm, topk_v, qw_v, qr_v, sem0, sem1, sem2):
    wid = lax.axis_index("s") * NC + lax.axis_index("c")
    base = wid * RPW
    kbase = base * K
    pltpu.sync_copy(topk_hbm.at[pl.ds(kbase, KPW)], topk_v)
    sems = (sem0, sem1, sem2)

    def fire(g, s):
        def _enq():
            pltpu.async_copy(
                q_hbm.at[pl.ds(base + g * QG, QG)], qr_v.at[s], sems[s])
        if isinstance(g, int):
            if g < NQG:
                _enq()
        else:
            pl.when(g < NQG)(_enq)

    def drain(s):
        pltpu.make_async_copy(
            q_hbm.at[pl.ds(base, QG)], qr_v.at[s], sems[s]).wait()

    def process(g, s):
        for n in range(QG):
            kb = g * QG * K + n * K
            tk0 = topk_v[pl.ds(kb, L)]
            tk1 = topk_v[pl.ds(kb + K - L, L)]
            row = jnp.full((L,), n, jnp.int32)
            g0 = plsc.load_gather(qr_v.at[s], [row, tk0])
            g1 = plsc.load_gather(qr_v.at[s], [row, tk1])
            qw_v[pl.ds(kb, L)] = g0
            qw_v[pl.ds(kb + K - L, L)] = g1

    for s in range(QNB):
        fire(s, s)

    def group_trip(i, carry):
        for s in range(QNB):
            g = QNB * i + s
            drain(s)
            process(g, s)
            fire(g + QNB, s)
        return carry
    lax.fori_loop(0, NQG // QNB, group_trip, 0)
    for g in range((NQG // QNB) * QNB, NQG):
        s = g % QNB
        drain(s)
        process(g, s)
    pltpu.sync_copy(qw_v, qw_hbm.at[pl.ds(kbase, KPW)])


@functools.lru_cache(maxsize=None)
def _qw_extract():
    return pl.kernel(
        _qw_body,
        out_type=jax.ShapeDtypeStruct((N * K,), jnp.float32),
        mesh=_mesh(),
        scratch_types=_QW_SCRATCH,
        compiler_params=pltpu.CompilerParams(needs_layout_passes=False),
    )


_AGG_SCRATCH = [
    pltpu.VMEM((KPW,), jnp.int32),
    pltpu.VMEM((KPW,), jnp.float32),
    pltpu.VMEM((RPW, HID), jnp.float32),
    pltpu.VMEM((2, KSUB, HID), jnp.float32),
    pltpu.VMEM((SUB, HID), jnp.float32),
    pltpu.SemaphoreType.DMA,
    pltpu.SemaphoreType.DMA,
]


def _agg_body(h_hbm, topk_hbm, qw_hbm, z_hbm,
              topk_v, qw_v, hown_v, rows_v, z_v, sem0, sem1):
    wid = lax.axis_index("s") * NC + lax.axis_index("c")
    base = wid * RPW
    kbase = base * K
    pltpu.sync_copy(topk_hbm.at[pl.ds(kbase, KPW)], topk_v)
    pltpu.sync_copy(qw_hbm.at[pl.ds(kbase, KPW)], qw_v)
    pltpu.sync_copy(h_hbm.at[pl.ds(base, RPW)], hown_v)
    sems = (sem0, sem1)

    def fire(c, s):
        if c >= NSUB:
            return
        cb = c * KSUB
        for j in range(NIC):
            pltpu.async_copy(
                h_hbm.at[topk_v.at[pl.ds(cb + j * IDX_CHUNK, IDX_CHUNK)]],
                rows_v.at[s, pl.ds(j * IDX_CHUNK, IDX_CHUNK)], sems[s])

    def drain(s):
        for j in range(NIC):
            pltpu.make_async_copy(
                h_hbm.at[pl.ds(0, IDX_CHUNK)],
                rows_v.at[s, pl.ds(j * IDX_CHUNK, IDX_CHUNK)], sems[s]).wait()

    fire(0, 0)
    fire(1, 1)
    for c in range(NSUB):
        s = c & 1
        drain(s)

        def node_step(n, carry, s=s, c=c):
            node = c * SUB + n
            kb = n * K
            qa = qw_v[pl.ds(node * K, L)]
            qb = qw_v[pl.ds(node * K + K - L, L)]
            accs = [hown_v[node, pl.ds(d * L, L)] for d in range(ND)]
            for k in range(K):
                if k < L:
                    b = jnp.broadcast_to(qa[k], (L,))
                else:
                    b = jnp.broadcast_to(qb[k - (K - L)], (L,))
                for d in range(ND):
                    accs[d] = accs[d] + b * rows_v[s, kb + k, pl.ds(d * L, L)]
            for d in range(ND):
                z_v[n, pl.ds(d * L, L)] = accs[d]
            return carry
        lax.fori_loop(0, SUB, node_step, 0)
        pltpu.sync_copy(z_v, z_hbm.at[pl.ds(base + c * SUB, SUB)])
        fire(c + 2, s)


@functools.lru_cache(maxsize=None)
def _aggregate():
    return pl.kernel(
        _agg_body,
        out_type=jax.ShapeDtypeStruct((N, HID), jnp.float32),
        mesh=_mesh(),
        scratch_types=_AGG_SCRATCH,
        compiler_params=pltpu.CompilerParams(
            needs_layout_passes=False, use_tc_tiling_on_sc=False),
    )


def kernel(X, Q, top_k, W1, b1, W2, b2, W3, b3, D1, bd1, D2, bd2):
    H = _encoder(X, W1, b1, W2, b2, W3, b3)
    topk_flat = top_k.reshape(-1)
    qw = _qw_extract()(Q, topk_flat)
    Z = _aggregate()(H, topk_flat, qw)
    X_rec = _decoder(Z, D1, bd1, D2, bd2)
    return (X_rec, Z)
